# baseline (speedup 1.0000x reference)
import functools

import jax
import jax.numpy as jnp
from jax import lax
from jax.experimental import pallas as pl
from jax.experimental.pallas import tpu as pltpu
from jax.experimental.pallas import tpu_sc as plsc

N = 10000
E = 320000
D = 128
DG = 16
NPAD = 10240
NC = 2
NS = 16
EPAD = 322560
CH = 40
TOTCH = EPAD // CH
NB = 6
FAST_CID = 0
CPW_FAST = 372
CPW_SLOW = 132
RPS = NPAD // NS


def _sc_agg_body(x, src, dst, zeros, ones16, zeros16, outs, sgi, dgi,
                 rows, zbuf, acc, acc_d, ones_v, dbuf,
                 sems, semi, semz, semd, *, with_deg):
    if with_deg:
        out, out_d = outs
    else:
        out = outs
    cid = lax.axis_index("c")
    sid = lax.axis_index("s")
    fast = cid == FAST_CID
    cb = jnp.where(fast, NS * CPW_SLOW + sid * CPW_FAST, sid * CPW_SLOW)
    ngrp = jnp.where(fast, CPW_FAST // NB, CPW_SLOW // NB)
    npair = jnp.where(fast, CPW_FAST // (2 * NB), CPW_SLOW // (2 * NB))
    base_r = sid * RPS

    _c1 = jax.named_scope("ph_pre")
    _c1.__enter__()
    pltpu.sync_copy(src.at[pl.ds(cb, NB)], sgi.at[0])
    pltpu.sync_copy(dst.at[pl.ds(cb, NB)], dgi.at[0])
    pltpu.make_async_copy(src.at[pl.ds(cb + NB, NB)], sgi.at[1], semi).start()
    pltpu.make_async_copy(dst.at[pl.ds(cb + NB, NB)], dgi.at[1], semi).start()
    for b in range(NB):
        pltpu.make_async_copy(x.at[sgi.at[0, b]], rows[b], sems[b]).start()

    pltpu.sync_copy(zeros.at[pl.ds(0, 16)], zbuf)
    if with_deg:
        pltpu.sync_copy(ones16, ones_v)
        pltpu.sync_copy(zeros16, dbuf)
        for i in range(RPS // CH):
            r = base_r + i * CH
            pltpu.make_async_copy(dbuf, acc_d.at[pl.ds(r, CH)], semd).start()
    nz = RPS // 16
    for w in range(0, nz, 8):
        for i in range(w, w + 8):
            r = base_r + i * 16
            pltpu.make_async_copy(zbuf, acc.at[pl.ds(r, 16)], semz).start()
        for i in range(w, w + 8):
            r = base_r + i * 16
            pltpu.make_async_copy(zbuf, acc.at[pl.ds(r, 16)], semz).wait()
    if with_deg:
        for i in range(RPS // CH):
            r = base_r + i * CH
            pltpu.make_async_copy(dbuf, acc_d.at[pl.ds(r, CH)], semd).wait()
    plsc.subcore_barrier()
    _c1.__exit__(None, None, None)
    _c2 = jax.named_scope("ph_edges")
    _c2.__enter__()

    def pair(p, carry):
        for sl in range(2):
            g = 2 * p + sl
            nsl = 1 - sl

            @pl.when(g + 1 < ngrp)
            def _():
                pltpu.make_async_copy(
                    src.at[pl.ds(cb + (g + 1) * NB, NB)], sgi.at[nsl],
                    semi).wait()
                pltpu.make_async_copy(
                    dst.at[pl.ds(cb + (g + 1) * NB, NB)], dgi.at[nsl],
                    semi).wait()

            for b in range(NB):
                pltpu.make_async_copy(
                    x.at[sgi.at[sl, b]], rows[b], sems[b]).wait()
                pltpu.sync_copy(rows[b], acc.at[dgi.at[sl, b]], add=True)
                if with_deg:
                    pltpu.sync_copy(ones_v, acc_d.at[dgi.at[sl, b]],
                                    add=True)

                @pl.when(g + 1 < ngrp)
                def _():
                    pltpu.make_async_copy(
                        x.at[sgi.at[nsl, b]], rows[b], sems[b]).start()

            @pl.when(g + 2 < ngrp)
            def _():
                pltpu.make_async_copy(
                    src.at[pl.ds(cb + (g + 2) * NB, NB)], sgi.at[sl],
                    semi).start()
                pltpu.make_async_copy(
                    dst.at[pl.ds(cb + (g + 2) * NB, NB)], dgi.at[sl],
                    semi).start()

        return carry

    lax.fori_loop(0, npair, pair, 0)
    plsc.subcore_barrier()
    _c2.__exit__(None, None, None)
    _c3 = jax.named_scope("ph_wb")
    _c3.__enter__()

    nwb = RPS // CH
    for i in range(nwb):
        b = i % NB
        if i >= NB:
            pltpu.make_async_copy(
                rows[b], out.at[cid, pl.ds(base_r + (i - NB) * CH, CH)],
                sems[b]).wait()
        pltpu.sync_copy(acc.at[pl.ds(base_r + i * CH, CH)], rows[b])
        pltpu.make_async_copy(
            rows[b], out.at[cid, pl.ds(base_r + i * CH, CH)],
            sems[b]).start()
    if with_deg:
        dbs = (dbuf, ones_v)
        dsems = (semd, semz)
        for i in range(nwb):
            b = i % 2
            if i >= 2:
                pltpu.make_async_copy(
                    dbs[b], out_d.at[cid, pl.ds(base_r + (i - 2) * CH, CH)],
                    dsems[b]).wait()
            pltpu.sync_copy(acc_d.at[pl.ds(base_r + i * CH, CH)], dbs[b])
            pltpu.make_async_copy(
                dbs[b], out_d.at[cid, pl.ds(base_r + i * CH, CH)],
                dsems[b]).start()
        for i in range(nwb - 2, nwb):
            b = i % 2
            pltpu.make_async_copy(
                dbs[b], out_d.at[cid, pl.ds(base_r + i * CH, CH)],
                dsems[b]).wait()
    for i in range(nwb - NB, nwb):
        b = i % NB
        pltpu.make_async_copy(
            rows[b], out.at[cid, pl.ds(base_r + i * CH, CH)],
            sems[b]).wait()
    _c3.__exit__(None, None, None)


def _sc_agg_body_deg(x, src, dst, zeros, ones16, zeros16, out, out_d, sgi,
                     dgi, *rest):
    rows, zbuf, acc, acc_d, ones_v, dbuf = rest[:NB], rest[NB], rest[NB + 1], rest[NB + 2], rest[NB + 3], rest[NB + 4]
    sems, semi, semz, semd = rest[NB + 5:2 * NB + 5], rest[2 * NB + 5], rest[2 * NB + 6], rest[2 * NB + 7]
    _sc_agg_body(x, src, dst, zeros, ones16, zeros16, (out, out_d), sgi,
                 dgi, rows, zbuf, acc, acc_d, ones_v, dbuf,
                 sems, semi, semz, semd, with_deg=True)


def _sc_agg_body_nodeg(x, src, dst, zeros, out, sgi, dgi, *rest):
    rows, zbuf, acc = rest[:NB], rest[NB], rest[NB + 1]
    sems, semi, semz = rest[NB + 2:2 * NB + 2], rest[2 * NB + 2], rest[2 * NB + 3]
    _sc_agg_body(x, src, dst, zeros, None, None, out, sgi, dgi, rows,
                 zbuf, acc, None, None, None, sems, semi, semz, None,
                 with_deg=False)


_sc_agg_deg = functools.partial(
    pl.kernel,
    mesh=plsc.VectorSubcoreMesh(core_axis_name="c", subcore_axis_name="s"),
    out_type=(jax.ShapeDtypeStruct((NC, NPAD, D), jnp.float32),
              jax.ShapeDtypeStruct((NC, NPAD, DG), jnp.float32)),
    scratch_types=[
        pltpu.VMEM((2, NB, CH), jnp.int32),
        pltpu.VMEM((2, NB, CH), jnp.int32),
    ] + [pltpu.VMEM((CH, D), jnp.float32) for _ in range(NB)] + [
        pltpu.VMEM((16, D), jnp.float32),
        pltpu.VMEM_SHARED((NPAD, D), jnp.float32),
        pltpu.VMEM_SHARED((NPAD, DG), jnp.float32),
        pltpu.VMEM((CH, DG), jnp.float32),
        pltpu.VMEM((CH, DG), jnp.float32),
    ] + [pltpu.SemaphoreType.DMA for _ in range(NB + 3)],
    compiler_params=pltpu.CompilerParams(use_tc_tiling_on_sc=False),
)(_sc_agg_body_deg)

_sc_agg_nodeg = functools.partial(
    pl.kernel,
    mesh=plsc.VectorSubcoreMesh(core_axis_name="c", subcore_axis_name="s"),
    out_type=jax.ShapeDtypeStruct((NC, NPAD, D), jnp.float32),
    scratch_types=[
        pltpu.VMEM((2, NB, CH), jnp.int32),
        pltpu.VMEM((2, NB, CH), jnp.int32),
    ] + [pltpu.VMEM((CH, D), jnp.float32) for _ in range(NB)] + [
        pltpu.VMEM((16, D), jnp.float32),
        pltpu.VMEM_SHARED((NPAD, D), jnp.float32),
    ] + [pltpu.SemaphoreType.DMA for _ in range(NB + 2)],
    compiler_params=pltpu.CompilerParams(use_tc_tiling_on_sc=False),
)(_sc_agg_body_nodeg)


def _tc_body(sums_ref, degs_ref, x_ref, ab_ref, bias_ref, out_ref, *, relu):
    s = sums_ref[0] + sums_ref[1]
    deg = jnp.maximum(degs_ref[0, :, :1] + degs_ref[1, :, :1], 1.0)
    m = jnp.concatenate([s / deg, x_ref[:]], axis=1)
    y = jnp.dot(m, ab_ref[:], preferred_element_type=jnp.float32)
    y = y + bias_ref[:]
    if relu:
        y = jnp.maximum(y, 0.0)
    out_ref[:] = y


def _dense_layer(sums, degs, x, ab_w, bias, relu):
    br = 400
    return pl.pallas_call(
        functools.partial(_tc_body, relu=relu),
        grid=(N // br,),
        in_specs=[
            pl.BlockSpec((NC, br, D), lambda i: (0, i, 0)),
            pl.BlockSpec((NC, br, DG), lambda i: (0, i, 0)),
            pl.BlockSpec((br, D), lambda i: (i, 0)),
            pl.BlockSpec((2 * D, D), lambda i: (0, 0)),
            pl.BlockSpec((1, D), lambda i: (0, 0)),
        ],
        out_specs=pl.BlockSpec((br, D), lambda i: (i, 0)),
        out_shape=jax.ShapeDtypeStruct((N, D), jnp.float32),
    )(sums, degs, x, ab_w, bias)


def kernel(x, edge_index, W0l, W0r, b0, bn_gamma, bn_beta, bn_mean, bn_var, W1l, W1r, b1):
    src = edge_index[0]
    dst = edge_index[1]

    g = bn_gamma / jnp.sqrt(bn_var + 1e-5)
    c = bn_beta - bn_mean * g
    ab0 = jnp.concatenate([W0l.T * g, W0r.T * g], axis=0)
    bias0 = (b0 * g + c)[None, :]
    ab1 = jnp.concatenate([W1l.T, W1r.T], axis=0)
    bias1 = b1[None, :]

    zeros = jnp.zeros((CH, D), jnp.float32)
    ones16 = jnp.ones((CH, DG), jnp.float32)
    zeros16 = jnp.zeros((CH, DG), jnp.float32)

    pad_e = EPAD - E
    src_c = jnp.concatenate(
        [src, jnp.zeros((pad_e,), jnp.int32)]).reshape(TOTCH, CH)
    dst_c = jnp.concatenate(
        [dst, jnp.full((pad_e,), NPAD - 1, jnp.int32)]).reshape(TOTCH, CH)

    sums0, degs = _sc_agg_deg(x, src_c, dst_c, zeros, ones16, zeros16)
    h = _dense_layer(sums0, degs, x, ab0, bias0, relu=True)
    sums1 = _sc_agg_nodeg(h, src_c, dst_c, zeros)
    logit = _dense_layer(sums1, degs, h, ab1, bias1, relu=False)

    return (logit, h)


# speedup vs baseline: 1.0374x; 1.0374x over previous
import functools

import jax
import jax.numpy as jnp
from jax import lax
from jax.experimental import pallas as pl
from jax.experimental.pallas import tpu as pltpu
from jax.experimental.pallas import tpu_sc as plsc

N = 10000
E = 320000
D = 128
DG = 16
NPAD = 10240
NC = 2
NS = 16
EPAD = 322560
CH = 80
TOTCH = EPAD // CH
NB = 3
FAST_CID = 0
CPW_FAST = 186
CPW_SLOW = 66
RPS = NPAD // NS


def _sc_agg_body(x, src, dst, zeros, ones16, zeros16, outs, sgi, dgi,
                 rows, zbuf, acc, acc_d, ones_v, dbuf,
                 sems, semi, semz, semd, *, with_deg):
    if with_deg:
        out, out_d = outs
    else:
        out = outs
    cid = lax.axis_index("c")
    sid = lax.axis_index("s")
    fast = cid == FAST_CID
    cb = jnp.where(fast, NS * CPW_SLOW + sid * CPW_FAST, sid * CPW_SLOW)
    ngrp = jnp.where(fast, CPW_FAST // NB, CPW_SLOW // NB)
    npair = jnp.where(fast, CPW_FAST // (2 * NB), CPW_SLOW // (2 * NB))
    base_r = sid * RPS

    _c1 = jax.named_scope("ph_pre")
    _c1.__enter__()
    pltpu.sync_copy(src.at[pl.ds(cb, NB)], sgi.at[0])
    pltpu.sync_copy(dst.at[pl.ds(cb, NB)], dgi.at[0])
    pltpu.make_async_copy(src.at[pl.ds(cb + NB, NB)], sgi.at[1], semi).start()
    pltpu.make_async_copy(dst.at[pl.ds(cb + NB, NB)], dgi.at[1], semi).start()
    for b in range(NB):
        pltpu.make_async_copy(x.at[sgi.at[0, b]], rows[b], sems[b]).start()

    pltpu.sync_copy(zeros.at[pl.ds(0, 16)], zbuf)
    if with_deg:
        pltpu.sync_copy(ones16, ones_v)
        pltpu.sync_copy(zeros16, dbuf)
        for i in range(RPS // CH):
            r = base_r + i * CH
            pltpu.make_async_copy(dbuf, acc_d.at[pl.ds(r, CH)], semd).start()
    nz = RPS // 16
    for w in range(0, nz, 8):
        for i in range(w, w + 8):
            r = base_r + i * 16
            pltpu.make_async_copy(zbuf, acc.at[pl.ds(r, 16)], semz).start()
        for i in range(w, w + 8):
            r = base_r + i * 16
            pltpu.make_async_copy(zbuf, acc.at[pl.ds(r, 16)], semz).wait()
    if with_deg:
        for i in range(RPS // CH):
            r = base_r + i * CH
            pltpu.make_async_copy(dbuf, acc_d.at[pl.ds(r, CH)], semd).wait()
    plsc.subcore_barrier()
    _c1.__exit__(None, None, None)
    _c2 = jax.named_scope("ph_edges")
    _c2.__enter__()

    def pair(p, carry):
        for sl in range(2):
            g = 2 * p + sl
            nsl = 1 - sl

            @pl.when(g + 1 < ngrp)
            def _():
                pltpu.make_async_copy(
                    src.at[pl.ds(cb + (g + 1) * NB, NB)], sgi.at[nsl],
                    semi).wait()
                pltpu.make_async_copy(
                    dst.at[pl.ds(cb + (g + 1) * NB, NB)], dgi.at[nsl],
                    semi).wait()

            for b in range(NB):
                pltpu.make_async_copy(
                    x.at[sgi.at[sl, b]], rows[b], sems[b]).wait()
                pltpu.sync_copy(rows[b], acc.at[dgi.at[sl, b]], add=True)
                if with_deg:
                    pltpu.sync_copy(ones_v, acc_d.at[dgi.at[sl, b]],
                                    add=True)

                @pl.when(g + 1 < ngrp)
                def _():
                    pltpu.make_async_copy(
                        x.at[sgi.at[nsl, b]], rows[b], sems[b]).start()

            @pl.when(g + 2 < ngrp)
            def _():
                pltpu.make_async_copy(
                    src.at[pl.ds(cb + (g + 2) * NB, NB)], sgi.at[sl],
                    semi).start()
                pltpu.make_async_copy(
                    dst.at[pl.ds(cb + (g + 2) * NB, NB)], dgi.at[sl],
                    semi).start()

        return carry

    lax.fori_loop(0, npair, pair, 0)
    plsc.subcore_barrier()
    _c2.__exit__(None, None, None)
    _c3 = jax.named_scope("ph_wb")
    _c3.__enter__()

    nwb = RPS // CH
    for i in range(nwb):
        b = i % NB
        if i >= NB:
            pltpu.make_async_copy(
                rows[b], out.at[cid, pl.ds(base_r + (i - NB) * CH, CH)],
                sems[b]).wait()
        pltpu.sync_copy(acc.at[pl.ds(base_r + i * CH, CH)], rows[b])
        pltpu.make_async_copy(
            rows[b], out.at[cid, pl.ds(base_r + i * CH, CH)],
            sems[b]).start()
    if with_deg:
        dbs = (dbuf, ones_v)
        dsems = (semd, semz)
        for i in range(nwb):
            b = i % 2
            if i >= 2:
                pltpu.make_async_copy(
                    dbs[b], out_d.at[cid, pl.ds(base_r + (i - 2) * CH, CH)],
                    dsems[b]).wait()
            pltpu.sync_copy(acc_d.at[pl.ds(base_r + i * CH, CH)], dbs[b])
            pltpu.make_async_copy(
                dbs[b], out_d.at[cid, pl.ds(base_r + i * CH, CH)],
                dsems[b]).start()
        for i in range(nwb - 2, nwb):
            b = i % 2
            pltpu.make_async_copy(
                dbs[b], out_d.at[cid, pl.ds(base_r + i * CH, CH)],
                dsems[b]).wait()
    for i in range(nwb - NB, nwb):
        b = i % NB
        pltpu.make_async_copy(
            rows[b], out.at[cid, pl.ds(base_r + i * CH, CH)],
            sems[b]).wait()
    _c3.__exit__(None, None, None)


def _sc_agg_body_deg(x, src, dst, zeros, ones16, zeros16, out, out_d, sgi,
                     dgi, *rest):
    rows, zbuf, acc, acc_d, ones_v, dbuf = rest[:NB], rest[NB], rest[NB + 1], rest[NB + 2], rest[NB + 3], rest[NB + 4]
    sems, semi, semz, semd = rest[NB + 5:2 * NB + 5], rest[2 * NB + 5], rest[2 * NB + 6], rest[2 * NB + 7]
    _sc_agg_body(x, src, dst, zeros, ones16, zeros16, (out, out_d), sgi,
                 dgi, rows, zbuf, acc, acc_d, ones_v, dbuf,
                 sems, semi, semz, semd, with_deg=True)


def _sc_agg_body_nodeg(x, src, dst, zeros, out, sgi, dgi, *rest):
    rows, zbuf, acc = rest[:NB], rest[NB], rest[NB + 1]
    sems, semi, semz = rest[NB + 2:2 * NB + 2], rest[2 * NB + 2], rest[2 * NB + 3]
    _sc_agg_body(x, src, dst, zeros, None, None, out, sgi, dgi, rows,
                 zbuf, acc, None, None, None, sems, semi, semz, None,
                 with_deg=False)


_sc_agg_deg = functools.partial(
    pl.kernel,
    mesh=plsc.VectorSubcoreMesh(core_axis_name="c", subcore_axis_name="s"),
    out_type=(jax.ShapeDtypeStruct((NC, NPAD, D), jnp.float32),
              jax.ShapeDtypeStruct((NC, NPAD, DG), jnp.float32)),
    scratch_types=[
        pltpu.VMEM((2, NB, CH), jnp.int32),
        pltpu.VMEM((2, NB, CH), jnp.int32),
    ] + [pltpu.VMEM((CH, D), jnp.float32) for _ in range(NB)] + [
        pltpu.VMEM((16, D), jnp.float32),
        pltpu.VMEM_SHARED((NPAD, D), jnp.float32),
        pltpu.VMEM_SHARED((NPAD, DG), jnp.float32),
        pltpu.VMEM((CH, DG), jnp.float32),
        pltpu.VMEM((CH, DG), jnp.float32),
    ] + [pltpu.SemaphoreType.DMA for _ in range(NB + 3)],
    compiler_params=pltpu.CompilerParams(use_tc_tiling_on_sc=False),
)(_sc_agg_body_deg)

_sc_agg_nodeg = functools.partial(
    pl.kernel,
    mesh=plsc.VectorSubcoreMesh(core_axis_name="c", subcore_axis_name="s"),
    out_type=jax.ShapeDtypeStruct((NC, NPAD, D), jnp.float32),
    scratch_types=[
        pltpu.VMEM((2, NB, CH), jnp.int32),
        pltpu.VMEM((2, NB, CH), jnp.int32),
    ] + [pltpu.VMEM((CH, D), jnp.float32) for _ in range(NB)] + [
        pltpu.VMEM((16, D), jnp.float32),
        pltpu.VMEM_SHARED((NPAD, D), jnp.float32),
    ] + [pltpu.SemaphoreType.DMA for _ in range(NB + 2)],
    compiler_params=pltpu.CompilerParams(use_tc_tiling_on_sc=False),
)(_sc_agg_body_nodeg)


def _tc_body(sums_ref, degs_ref, x_ref, ab_ref, bias_ref, out_ref, *, relu):
    s = sums_ref[0] + sums_ref[1]
    deg = jnp.maximum(degs_ref[0, :, :1] + degs_ref[1, :, :1], 1.0)
    m = jnp.concatenate([s / deg, x_ref[:]], axis=1)
    y = jnp.dot(m, ab_ref[:], preferred_element_type=jnp.float32)
    y = y + bias_ref[:]
    if relu:
        y = jnp.maximum(y, 0.0)
    out_ref[:] = y


def _dense_layer(sums, degs, x, ab_w, bias, relu):
    br = 400
    return pl.pallas_call(
        functools.partial(_tc_body, relu=relu),
        grid=(N // br,),
        in_specs=[
            pl.BlockSpec((NC, br, D), lambda i: (0, i, 0)),
            pl.BlockSpec((NC, br, DG), lambda i: (0, i, 0)),
            pl.BlockSpec((br, D), lambda i: (i, 0)),
            pl.BlockSpec((2 * D, D), lambda i: (0, 0)),
            pl.BlockSpec((1, D), lambda i: (0, 0)),
        ],
        out_specs=pl.BlockSpec((br, D), lambda i: (i, 0)),
        out_shape=jax.ShapeDtypeStruct((N, D), jnp.float32),
    )(sums, degs, x, ab_w, bias)


def kernel(x, edge_index, W0l, W0r, b0, bn_gamma, bn_beta, bn_mean, bn_var, W1l, W1r, b1):
    src = edge_index[0]
    dst = edge_index[1]

    g = bn_gamma / jnp.sqrt(bn_var + 1e-5)
    c = bn_beta - bn_mean * g
    ab0 = jnp.concatenate([W0l.T * g, W0r.T * g], axis=0)
    bias0 = (b0 * g + c)[None, :]
    ab1 = jnp.concatenate([W1l.T, W1r.T], axis=0)
    bias1 = b1[None, :]

    zeros = jnp.zeros((CH, D), jnp.float32)
    ones16 = jnp.ones((CH, DG), jnp.float32)
    zeros16 = jnp.zeros((CH, DG), jnp.float32)

    pad_e = EPAD - E
    src_c = jnp.concatenate(
        [src, jnp.zeros((pad_e,), jnp.int32)]).reshape(TOTCH, CH)
    dst_c = jnp.concatenate(
        [dst, jnp.full((pad_e,), NPAD - 1, jnp.int32)]).reshape(TOTCH, CH)

    sums0, degs = _sc_agg_deg(x, src_c, dst_c, zeros, ones16, zeros16)
    h = _dense_layer(sums0, degs, x, ab0, bias0, relu=True)
    sums1 = _sc_agg_nodeg(h, src_c, dst_c, zeros)
    logit = _dense_layer(sums1, degs, h, ab1, bias1, relu=False)

    return (logit, h)


# speedup vs baseline: 1.1541x; 1.1125x over previous
import functools

import jax
import jax.numpy as jnp
from jax import lax
from jax.experimental import pallas as pl
from jax.experimental.pallas import tpu as pltpu
from jax.experimental.pallas import tpu_sc as plsc

N = 10000
E = 320000
D = 128
DG = 16
NPAD = 10240
NC = 2
NS = 16
EPAD = 322560
CH = 80
TOTCH = EPAD // CH
NB = 3
FAST_CID = 0
CPW_FAST = 126
CPW_SLOW = 126
RPS = NPAD // NS


def _sc_agg_body(x, src, dst, zeros, ones16, zeros16, outs, sgi, dgi,
                 rows, zbuf, acc, acc_d, ones_v, dbuf,
                 sems, semi, semz, semd, *, with_deg):
    if with_deg:
        out, out_d = outs
    else:
        out = outs
    cid = lax.axis_index("c")
    sid = lax.axis_index("s")
    fast = cid == FAST_CID
    cb = jnp.where(fast, NS * CPW_SLOW + sid * CPW_FAST, sid * CPW_SLOW)
    ngrp = jnp.where(fast, CPW_FAST // NB, CPW_SLOW // NB)
    npair = jnp.where(fast, CPW_FAST // (2 * NB), CPW_SLOW // (2 * NB))
    base_r = sid * RPS

    _c1 = jax.named_scope("ph_pre")
    _c1.__enter__()
    pltpu.sync_copy(src.at[pl.ds(cb, NB)], sgi.at[0])
    pltpu.sync_copy(dst.at[pl.ds(cb, NB)], dgi.at[0])
    pltpu.make_async_copy(src.at[pl.ds(cb + NB, NB)], sgi.at[1], semi).start()
    pltpu.make_async_copy(dst.at[pl.ds(cb + NB, NB)], dgi.at[1], semi).start()
    for b in range(NB):
        pltpu.make_async_copy(x.at[sgi.at[0, b]], rows[b], sems[b]).start()

    pltpu.sync_copy(zeros.at[pl.ds(0, 16)], zbuf)
    if with_deg:
        pltpu.sync_copy(ones16, ones_v)
        pltpu.sync_copy(zeros16, dbuf)
        for i in range(RPS // CH):
            r = base_r + i * CH
            pltpu.make_async_copy(dbuf, acc_d.at[pl.ds(r, CH)], semd).start()
    nz = RPS // 16
    for w in range(0, nz, 8):
        for i in range(w, w + 8):
            r = base_r + i * 16
            pltpu.make_async_copy(zbuf, acc.at[pl.ds(r, 16)], semz).start()
        for i in range(w, w + 8):
            r = base_r + i * 16
            pltpu.make_async_copy(zbuf, acc.at[pl.ds(r, 16)], semz).wait()
    if with_deg:
        for i in range(RPS // CH):
            r = base_r + i * CH
            pltpu.make_async_copy(dbuf, acc_d.at[pl.ds(r, CH)], semd).wait()
    plsc.subcore_barrier()
    _c1.__exit__(None, None, None)
    _c2 = jax.named_scope("ph_edges")
    _c2.__enter__()

    def pair(p, carry):
        for sl in range(2):
            g = 2 * p + sl
            nsl = 1 - sl

            @pl.when(g + 1 < ngrp)
            def _():
                pltpu.make_async_copy(
                    src.at[pl.ds(cb + (g + 1) * NB, NB)], sgi.at[nsl],
                    semi).wait()
                pltpu.make_async_copy(
                    dst.at[pl.ds(cb + (g + 1) * NB, NB)], dgi.at[nsl],
                    semi).wait()

            for b in range(NB):
                pltpu.make_async_copy(
                    x.at[sgi.at[sl, b]], rows[b], sems[b]).wait()
                pltpu.sync_copy(rows[b], acc.at[dgi.at[sl, b]], add=True)
                if with_deg:
                    pltpu.sync_copy(ones_v, acc_d.at[dgi.at[sl, b]],
                                    add=True)

                @pl.when(g + 1 < ngrp)
                def _():
                    pltpu.make_async_copy(
                        x.at[sgi.at[nsl, b]], rows[b], sems[b]).start()

            @pl.when(g + 2 < ngrp)
            def _():
                pltpu.make_async_copy(
                    src.at[pl.ds(cb + (g + 2) * NB, NB)], sgi.at[sl],
                    semi).start()
                pltpu.make_async_copy(
                    dst.at[pl.ds(cb + (g + 2) * NB, NB)], dgi.at[sl],
                    semi).start()

        return carry

    lax.fori_loop(0, npair, pair, 0)
    plsc.subcore_barrier()
    _c2.__exit__(None, None, None)
    _c3 = jax.named_scope("ph_wb")
    _c3.__enter__()

    nwb = RPS // CH
    for i in range(nwb):
        b = i % NB
        if i >= NB:
            pltpu.make_async_copy(
                rows[b], out.at[cid, pl.ds(base_r + (i - NB) * CH, CH)],
                sems[b]).wait()
        pltpu.sync_copy(acc.at[pl.ds(base_r + i * CH, CH)], rows[b])
        pltpu.make_async_copy(
            rows[b], out.at[cid, pl.ds(base_r + i * CH, CH)],
            sems[b]).start()
    if with_deg:
        dbs = (dbuf, ones_v)
        dsems = (semd, semz)
        for i in range(nwb):
            b = i % 2
            if i >= 2:
                pltpu.make_async_copy(
                    dbs[b], out_d.at[cid, pl.ds(base_r + (i - 2) * CH, CH)],
                    dsems[b]).wait()
            pltpu.sync_copy(acc_d.at[pl.ds(base_r + i * CH, CH)], dbs[b])
            pltpu.make_async_copy(
                dbs[b], out_d.at[cid, pl.ds(base_r + i * CH, CH)],
                dsems[b]).start()
        for i in range(nwb - 2, nwb):
            b = i % 2
            pltpu.make_async_copy(
                dbs[b], out_d.at[cid, pl.ds(base_r + i * CH, CH)],
                dsems[b]).wait()
    for i in range(nwb - NB, nwb):
        b = i % NB
        pltpu.make_async_copy(
            rows[b], out.at[cid, pl.ds(base_r + i * CH, CH)],
            sems[b]).wait()
    _c3.__exit__(None, None, None)


def _sc_agg_body_deg(x, src, dst, zeros, ones16, zeros16, out, out_d, sgi,
                     dgi, *rest):
    rows, zbuf, acc, acc_d, ones_v, dbuf = rest[:NB], rest[NB], rest[NB + 1], rest[NB + 2], rest[NB + 3], rest[NB + 4]
    sems, semi, semz, semd = rest[NB + 5:2 * NB + 5], rest[2 * NB + 5], rest[2 * NB + 6], rest[2 * NB + 7]
    _sc_agg_body(x, src, dst, zeros, ones16, zeros16, (out, out_d), sgi,
                 dgi, rows, zbuf, acc, acc_d, ones_v, dbuf,
                 sems, semi, semz, semd, with_deg=True)


def _sc_agg_body_nodeg(x, src, dst, zeros, out, sgi, dgi, *rest):
    rows, zbuf, acc = rest[:NB], rest[NB], rest[NB + 1]
    sems, semi, semz = rest[NB + 2:2 * NB + 2], rest[2 * NB + 2], rest[2 * NB + 3]
    _sc_agg_body(x, src, dst, zeros, None, None, out, sgi, dgi, rows,
                 zbuf, acc, None, None, None, sems, semi, semz, None,
                 with_deg=False)


_sc_agg_deg = functools.partial(
    pl.kernel,
    mesh=plsc.VectorSubcoreMesh(core_axis_name="c", subcore_axis_name="s"),
    out_type=(jax.ShapeDtypeStruct((NC, NPAD, D), jnp.float32),
              jax.ShapeDtypeStruct((NC, NPAD, DG), jnp.float32)),
    scratch_types=[
        pltpu.VMEM((2, NB, CH), jnp.int32),
        pltpu.VMEM((2, NB, CH), jnp.int32),
    ] + [pltpu.VMEM((CH, D), jnp.float32) for _ in range(NB)] + [
        pltpu.VMEM((16, D), jnp.float32),
        pltpu.VMEM_SHARED((NPAD, D), jnp.float32),
        pltpu.VMEM_SHARED((NPAD, DG), jnp.float32),
        pltpu.VMEM((CH, DG), jnp.float32),
        pltpu.VMEM((CH, DG), jnp.float32),
    ] + [pltpu.SemaphoreType.DMA for _ in range(NB + 3)],
    compiler_params=pltpu.CompilerParams(use_tc_tiling_on_sc=False),
)(_sc_agg_body_deg)

_sc_agg_nodeg = functools.partial(
    pl.kernel,
    mesh=plsc.VectorSubcoreMesh(core_axis_name="c", subcore_axis_name="s"),
    out_type=jax.ShapeDtypeStruct((NC, NPAD, D), jnp.float32),
    scratch_types=[
        pltpu.VMEM((2, NB, CH), jnp.int32),
        pltpu.VMEM((2, NB, CH), jnp.int32),
    ] + [pltpu.VMEM((CH, D), jnp.float32) for _ in range(NB)] + [
        pltpu.VMEM((16, D), jnp.float32),
        pltpu.VMEM_SHARED((NPAD, D), jnp.float32),
    ] + [pltpu.SemaphoreType.DMA for _ in range(NB + 2)],
    compiler_params=pltpu.CompilerParams(use_tc_tiling_on_sc=False),
)(_sc_agg_body_nodeg)


def _tc_body(sums_ref, degs_ref, x_ref, ab_ref, bias_ref, out_ref, *, relu):
    s = sums_ref[0] + sums_ref[1]
    deg = jnp.maximum(degs_ref[0, :, :1] + degs_ref[1, :, :1], 1.0)
    m = jnp.concatenate([s / deg, x_ref[:]], axis=1)
    y = jnp.dot(m, ab_ref[:], preferred_element_type=jnp.float32)
    y = y + bias_ref[:]
    if relu:
        y = jnp.maximum(y, 0.0)
    out_ref[:] = y


def _dense_layer(sums, degs, x, ab_w, bias, relu):
    br = 400
    return pl.pallas_call(
        functools.partial(_tc_body, relu=relu),
        grid=(N // br,),
        in_specs=[
            pl.BlockSpec((NC, br, D), lambda i: (0, i, 0)),
            pl.BlockSpec((NC, br, DG), lambda i: (0, i, 0)),
            pl.BlockSpec((br, D), lambda i: (i, 0)),
            pl.BlockSpec((2 * D, D), lambda i: (0, 0)),
            pl.BlockSpec((1, D), lambda i: (0, 0)),
        ],
        out_specs=pl.BlockSpec((br, D), lambda i: (i, 0)),
        out_shape=jax.ShapeDtypeStruct((N, D), jnp.float32),
    )(sums, degs, x, ab_w, bias)


def kernel(x, edge_index, W0l, W0r, b0, bn_gamma, bn_beta, bn_mean, bn_var, W1l, W1r, b1):
    src = edge_index[0]
    dst = edge_index[1]

    g = bn_gamma / jnp.sqrt(bn_var + 1e-5)
    c = bn_beta - bn_mean * g
    ab0 = jnp.concatenate([W0l.T * g, W0r.T * g], axis=0)
    bias0 = (b0 * g + c)[None, :]
    ab1 = jnp.concatenate([W1l.T, W1r.T], axis=0)
    bias1 = b1[None, :]

    zeros = jnp.zeros((CH, D), jnp.float32)
    ones16 = jnp.ones((CH, DG), jnp.float32)
    zeros16 = jnp.zeros((CH, DG), jnp.float32)

    pad_e = EPAD - E
    src_c = jnp.concatenate(
        [src, jnp.zeros((pad_e,), jnp.int32)]).reshape(TOTCH, CH)
    pad_dst = N + (jnp.arange(pad_e, dtype=jnp.int32) % (NPAD - N))
    dst_c = jnp.concatenate([dst, pad_dst]).reshape(TOTCH, CH)

    sums0, degs = _sc_agg_deg(x, src_c, dst_c, zeros, ones16, zeros16)
    h = _dense_layer(sums0, degs, x, ab0, bias0, relu=True)
    sums1 = _sc_agg_nodeg(h, src_c, dst_c, zeros)
    logit = _dense_layer(sums1, degs, h, ab1, bias1, relu=False)

    return (logit, h)


# speedup vs baseline: 2.0288x; 1.7579x over previous
import functools

import jax
import jax.numpy as jnp
from jax import lax
from jax.experimental import pallas as pl
from jax.experimental.pallas import tpu as pltpu
from jax.experimental.pallas import tpu_sc as plsc

N = 10000
E = 320000
D = 128
DG = 16
NPAD = 10240
NC = 2
NS = 16
EPAD = 322560
CH = 80
TOTCH = EPAD // CH
NB = 3
FAST_CID = 0
CPW_FAST = 126
CPW_SLOW = 126
RPS = NPAD // NS


def _sc_agg_body(x, src, dst, zeros, ones16, zeros16, outs, sgi, dgi,
                 rows, zbuf, acc, acc_d, ones_v, dbuf,
                 sems, semi, semz, semd, *, with_deg):
    if with_deg:
        out, out_d = outs
    else:
        out = outs
    cid = lax.axis_index("c")
    sid = lax.axis_index("s")
    fast = cid == FAST_CID
    cb = jnp.where(fast, NS * CPW_SLOW + sid * CPW_FAST, sid * CPW_SLOW)
    ngrp = jnp.where(fast, CPW_FAST // NB, CPW_SLOW // NB)
    npair = jnp.where(fast, CPW_FAST // (2 * NB), CPW_SLOW // (2 * NB))
    base_r = sid * RPS

    _c1 = jax.named_scope("ph_pre")
    _c1.__enter__()
    pltpu.sync_copy(src.at[pl.ds(cb, NB)], sgi.at[0])
    pltpu.sync_copy(dst.at[pl.ds(cb, NB)], dgi.at[0])
    pltpu.make_async_copy(src.at[pl.ds(cb + NB, NB)], sgi.at[1], semi).start()
    pltpu.make_async_copy(dst.at[pl.ds(cb + NB, NB)], dgi.at[1], semi).start()
    for b in range(NB):
        pltpu.make_async_copy(x.at[sgi.at[0, b]], rows[b], sems[b]).start()

    pltpu.sync_copy(zeros.at[pl.ds(0, 16)], zbuf)
    if with_deg:
        pltpu.sync_copy(ones16, ones_v)
        pltpu.sync_copy(zeros16, dbuf)
        for i in range(RPS // CH):
            r = base_r + i * CH
            pltpu.make_async_copy(dbuf, acc_d.at[pl.ds(r, CH)], semd).start()
    nz = RPS // 16
    for w in range(0, nz, 8):
        for i in range(w, w + 8):
            r = base_r + i * 16
            pltpu.make_async_copy(zbuf, acc.at[pl.ds(r, 16)], semz).start()
        for i in range(w, w + 8):
            r = base_r + i * 16
            pltpu.make_async_copy(zbuf, acc.at[pl.ds(r, 16)], semz).wait()
    if with_deg:
        for i in range(RPS // CH):
            r = base_r + i * CH
            pltpu.make_async_copy(dbuf, acc_d.at[pl.ds(r, CH)], semd).wait()
    plsc.subcore_barrier()
    _c1.__exit__(None, None, None)
    _c2 = jax.named_scope("ph_edges")
    _c2.__enter__()

    def pair(p, carry):
        for sl in range(2):
            g = 2 * p + sl
            nsl = 1 - sl

            @pl.when(g + 1 < ngrp)
            def _():
                pltpu.make_async_copy(
                    src.at[pl.ds(cb + (g + 1) * NB, NB)], sgi.at[nsl],
                    semi).wait()
                pltpu.make_async_copy(
                    dst.at[pl.ds(cb + (g + 1) * NB, NB)], dgi.at[nsl],
                    semi).wait()

            for b in range(NB):
                pltpu.make_async_copy(
                    x.at[sgi.at[sl, b]], rows[b], sems[b]).wait()
                pltpu.sync_copy(rows[b], acc.at[dgi.at[sl, b]], add=True)
                if with_deg:
                    pltpu.sync_copy(ones_v, acc_d.at[dgi.at[sl, b]],
                                    add=True)

                @pl.when(g + 1 < ngrp)
                def _():
                    pltpu.make_async_copy(
                        x.at[sgi.at[nsl, b]], rows[b], sems[b]).start()

            @pl.when(g + 2 < ngrp)
            def _():
                pltpu.make_async_copy(
                    src.at[pl.ds(cb + (g + 2) * NB, NB)], sgi.at[sl],
                    semi).start()
                pltpu.make_async_copy(
                    dst.at[pl.ds(cb + (g + 2) * NB, NB)], dgi.at[sl],
                    semi).start()

        return carry

    lax.fori_loop(0, npair, pair, 0)
    plsc.subcore_barrier()
    _c2.__exit__(None, None, None)
    _c3 = jax.named_scope("ph_wb")
    _c3.__enter__()

    nwb = RPS // CH
    for i in range(nwb):
        b = i % NB
        if i >= NB:
            pltpu.make_async_copy(
                rows[b], out.at[cid, pl.ds(base_r + (i - NB) * CH, CH)],
                sems[b]).wait()
        pltpu.sync_copy(acc.at[pl.ds(base_r + i * CH, CH)], rows[b])
        pltpu.make_async_copy(
            rows[b], out.at[cid, pl.ds(base_r + i * CH, CH)],
            sems[b]).start()
    if with_deg:
        dbs = (dbuf, ones_v)
        dsems = (semd, semz)
        for i in range(nwb):
            b = i % 2
            if i >= 2:
                pltpu.make_async_copy(
                    dbs[b], out_d.at[cid, pl.ds(base_r + (i - 2) * CH, CH)],
                    dsems[b]).wait()
            pltpu.sync_copy(acc_d.at[pl.ds(base_r + i * CH, CH)], dbs[b])
            pltpu.make_async_copy(
                dbs[b], out_d.at[cid, pl.ds(base_r + i * CH, CH)],
                dsems[b]).start()
        for i in range(nwb - 2, nwb):
            b = i % 2
            pltpu.make_async_copy(
                dbs[b], out_d.at[cid, pl.ds(base_r + i * CH, CH)],
                dsems[b]).wait()
    for i in range(nwb - NB, nwb):
        b = i % NB
        pltpu.make_async_copy(
            rows[b], out.at[cid, pl.ds(base_r + i * CH, CH)],
            sems[b]).wait()
    _c3.__exit__(None, None, None)


def _sc_agg_body_deg(x, src, dst, zeros, ones16, zeros16, out, out_d, sgi,
                     dgi, *rest):
    rows, zbuf, acc, acc_d, ones_v, dbuf = rest[:NB], rest[NB], rest[NB + 1], rest[NB + 2], rest[NB + 3], rest[NB + 4]
    sems, semi, semz, semd = rest[NB + 5:2 * NB + 5], rest[2 * NB + 5], rest[2 * NB + 6], rest[2 * NB + 7]
    _sc_agg_body(x, src, dst, zeros, ones16, zeros16, (out, out_d), sgi,
                 dgi, rows, zbuf, acc, acc_d, ones_v, dbuf,
                 sems, semi, semz, semd, with_deg=True)


def _sc_agg_body_nodeg(x, src, dst, zeros, out, sgi, dgi, *rest):
    rows, zbuf, acc = rest[:NB], rest[NB], rest[NB + 1]
    sems, semi, semz = rest[NB + 2:2 * NB + 2], rest[2 * NB + 2], rest[2 * NB + 3]
    _sc_agg_body(x, src, dst, zeros, None, None, out, sgi, dgi, rows,
                 zbuf, acc, None, None, None, sems, semi, semz, None,
                 with_deg=False)


_sc_agg_deg = functools.partial(
    pl.kernel,
    mesh=plsc.VectorSubcoreMesh(core_axis_name="c", subcore_axis_name="s"),
    out_type=(jax.ShapeDtypeStruct((NC, NPAD, D), jnp.float32),
              jax.ShapeDtypeStruct((NC, NPAD, DG), jnp.float32)),
    scratch_types=[
        pltpu.VMEM((2, NB, CH), jnp.int32),
        pltpu.VMEM((2, NB, CH), jnp.int32),
    ] + [pltpu.VMEM((CH, D), jnp.float32) for _ in range(NB)] + [
        pltpu.VMEM((16, D), jnp.float32),
        pltpu.VMEM_SHARED((NPAD, D), jnp.float32),
        pltpu.VMEM_SHARED((NPAD, DG), jnp.float32),
        pltpu.VMEM((CH, DG), jnp.float32),
        pltpu.VMEM((CH, DG), jnp.float32),
    ] + [pltpu.SemaphoreType.DMA for _ in range(NB + 3)],
    compiler_params=pltpu.CompilerParams(use_tc_tiling_on_sc=False),
)(_sc_agg_body_deg)

_sc_agg_nodeg = functools.partial(
    pl.kernel,
    mesh=plsc.VectorSubcoreMesh(core_axis_name="c", subcore_axis_name="s"),
    out_type=jax.ShapeDtypeStruct((NC, NPAD, D), jnp.float32),
    scratch_types=[
        pltpu.VMEM((2, NB, CH), jnp.int32),
        pltpu.VMEM((2, NB, CH), jnp.int32),
    ] + [pltpu.VMEM((CH, D), jnp.float32) for _ in range(NB)] + [
        pltpu.VMEM((16, D), jnp.float32),
        pltpu.VMEM_SHARED((NPAD, D), jnp.float32),
    ] + [pltpu.SemaphoreType.DMA for _ in range(NB + 2)],
    compiler_params=pltpu.CompilerParams(use_tc_tiling_on_sc=False),
)(_sc_agg_body_nodeg)


def _tc_body(sums_ref, degs_ref, x_ref, ab_ref, bias_ref, out_ref, *, relu):
    s = sums_ref[0] + sums_ref[1]
    deg = jnp.maximum(degs_ref[0, :, :1] + degs_ref[1, :, :1], 1.0)
    m = jnp.concatenate([s / deg, x_ref[:]], axis=1)
    y = jnp.dot(m, ab_ref[:], preferred_element_type=jnp.float32)
    y = y + bias_ref[:]
    if relu:
        y = jnp.maximum(y, 0.0)
    out_ref[:] = y


def _dense_layer(sums, degs, x, ab_w, bias, relu):
    br = 400
    return pl.pallas_call(
        functools.partial(_tc_body, relu=relu),
        grid=(N // br,),
        in_specs=[
            pl.BlockSpec((NC, br, D), lambda i: (0, i, 0)),
            pl.BlockSpec((NC, br, DG), lambda i: (0, i, 0)),
            pl.BlockSpec((br, D), lambda i: (i, 0)),
            pl.BlockSpec((2 * D, D), lambda i: (0, 0)),
            pl.BlockSpec((1, D), lambda i: (0, 0)),
        ],
        out_specs=pl.BlockSpec((br, D), lambda i: (i, 0)),
        out_shape=jax.ShapeDtypeStruct((N, D), jnp.float32),
    )(sums, degs, x, ab_w, bias)


def kernel(x, edge_index, W0l, W0r, b0, bn_gamma, bn_beta, bn_mean, bn_var, W1l, W1r, b1):
    src = edge_index[0]
    dst = edge_index[1]

    g = bn_gamma / jnp.sqrt(bn_var + 1e-5)
    c = bn_beta - bn_mean * g
    ab0 = jnp.concatenate([W0l.T * g, W0r.T * g], axis=0)
    bias0 = (b0 * g + c)[None, :]
    ab1 = jnp.concatenate([W1l.T, W1r.T], axis=0)
    bias1 = b1[None, :]

    zeros = jnp.zeros((CH, D), jnp.float32)
    ones16 = jnp.ones((CH, DG), jnp.float32)
    zeros16 = jnp.zeros((CH, DG), jnp.float32)

    pad_e = EPAD - E
    pad_src = jnp.arange(pad_e, dtype=jnp.int32) % N
    src_c = jnp.concatenate([src, pad_src]).reshape(TOTCH, CH)
    pad_dst = N + (jnp.arange(pad_e, dtype=jnp.int32) % (NPAD - N))
    dst_c = jnp.concatenate([dst, pad_dst]).reshape(TOTCH, CH)

    sums0, degs = _sc_agg_deg(x, src_c, dst_c, zeros, ones16, zeros16)
    h = _dense_layer(sums0, degs, x, ab0, bias0, relu=True)
    sums1 = _sc_agg_nodeg(h, src_c, dst_c, zeros)
    logit = _dense_layer(sums1, degs, h, ab1, bias1, relu=False)

    return (logit, h)


# speedup vs baseline: 2.1384x; 1.0540x over previous
import functools

import jax
import jax.numpy as jnp
from jax import lax
from jax.experimental import pallas as pl
from jax.experimental.pallas import tpu as pltpu
from jax.experimental.pallas import tpu_sc as plsc

N = 10000
E = 320000
D = 128
DG = 16
NPAD = 10240
NC = 2
NS = 16
EPAD = 322560
CH = 80
TOTCH = EPAD // CH
NB = 3
FAST_CID = 0
CPW_FAST = 126
CPW_SLOW = 126
RPS = NPAD // NS


def _sc_agg_body(x, src, dst, zeros, ones16, zeros16, outs, sgi, dgi,
                 rows, zbuf, acc, acc_d, ones_v, dbuf,
                 sems, semi, semz, semd, *, with_deg):
    if with_deg:
        out, out_d = outs
    else:
        out = outs
    cid = lax.axis_index("c")
    sid = lax.axis_index("s")
    fast = cid == FAST_CID
    cb = jnp.where(fast, NS * CPW_SLOW + sid * CPW_FAST, sid * CPW_SLOW)
    ngrp = jnp.where(fast, CPW_FAST // NB, CPW_SLOW // NB)
    npair = jnp.where(fast, CPW_FAST // (2 * NB), CPW_SLOW // (2 * NB))
    base_r = sid * RPS

    _c1 = jax.named_scope("ph_pre")
    _c1.__enter__()
    pltpu.sync_copy(src.at[pl.ds(cb, NB)], sgi.at[0])
    pltpu.sync_copy(dst.at[pl.ds(cb, NB)], dgi.at[0])
    pltpu.make_async_copy(src.at[pl.ds(cb + NB, NB)], sgi.at[1], semi).start()
    pltpu.make_async_copy(dst.at[pl.ds(cb + NB, NB)], dgi.at[1], semi).start()
    for b in range(NB):
        pltpu.make_async_copy(x.at[sgi.at[0, b]], rows[b], sems[b]).start()

    pltpu.sync_copy(zeros.at[pl.ds(0, 16)], zbuf)
    if with_deg:
        pltpu.sync_copy(ones16, ones_v)
        pltpu.sync_copy(zeros16, dbuf)
        for i in range(RPS // CH):
            r = base_r + i * CH
            pltpu.make_async_copy(dbuf, acc_d.at[pl.ds(r, CH)], semd).start()
    nz = RPS // 16
    for w in range(0, nz, 8):
        for i in range(w, w + 8):
            r = base_r + i * 16
            pltpu.make_async_copy(zbuf, acc.at[pl.ds(r, 16)], semz).start()
        for i in range(w, w + 8):
            r = base_r + i * 16
            pltpu.make_async_copy(zbuf, acc.at[pl.ds(r, 16)], semz).wait()
    if with_deg:
        for i in range(RPS // CH):
            r = base_r + i * CH
            pltpu.make_async_copy(dbuf, acc_d.at[pl.ds(r, CH)], semd).wait()
    plsc.subcore_barrier()
    _c1.__exit__(None, None, None)
    _c2 = jax.named_scope("ph_edges")
    _c2.__enter__()

    def pair(p, carry):
        for sl in range(2):
            g = 2 * p + sl
            nsl = 1 - sl

            @pl.when(g + 1 < ngrp)
            def _():
                pltpu.make_async_copy(
                    src.at[pl.ds(cb + (g + 1) * NB, NB)], sgi.at[nsl],
                    semi).wait()
                pltpu.make_async_copy(
                    dst.at[pl.ds(cb + (g + 1) * NB, NB)], dgi.at[nsl],
                    semi).wait()

            for b in range(NB):
                pltpu.make_async_copy(
                    x.at[sgi.at[sl, b]], rows[b], sems[b]).wait()
                pltpu.sync_copy(rows[b], acc.at[dgi.at[sl, b]], add=True)
                if with_deg:
                    pltpu.sync_copy(ones_v, acc_d.at[dgi.at[sl, b]],
                                    add=True)

                @pl.when(g + 1 < ngrp)
                def _():
                    pltpu.make_async_copy(
                        x.at[sgi.at[nsl, b]], rows[b], sems[b]).start()

            @pl.when(g + 2 < ngrp)
            def _():
                pltpu.make_async_copy(
                    src.at[pl.ds(cb + (g + 2) * NB, NB)], sgi.at[sl],
                    semi).start()
                pltpu.make_async_copy(
                    dst.at[pl.ds(cb + (g + 2) * NB, NB)], dgi.at[sl],
                    semi).start()

        return carry

    lax.fori_loop(0, npair, pair, 0)
    plsc.subcore_barrier()
    _c2.__exit__(None, None, None)
    _c3 = jax.named_scope("ph_wb")
    _c3.__enter__()

    nwb = RPS // CH
    for i in range(nwb):
        b = i % NB
        if i >= NB:
            pltpu.make_async_copy(
                rows[b], out.at[cid, pl.ds(base_r + (i - NB) * CH, CH)],
                sems[b]).wait()
        pltpu.sync_copy(acc.at[pl.ds(base_r + i * CH, CH)], rows[b])
        pltpu.make_async_copy(
            rows[b], out.at[cid, pl.ds(base_r + i * CH, CH)],
            sems[b]).start()
    if with_deg:
        dbs = (dbuf, ones_v)
        dsems = (semd, semz)
        for i in range(nwb):
            b = i % 2
            if i >= 2:
                pltpu.make_async_copy(
                    dbs[b], out_d.at[cid, pl.ds(base_r + (i - 2) * CH, CH)],
                    dsems[b]).wait()
            pltpu.sync_copy(acc_d.at[pl.ds(base_r + i * CH, CH)], dbs[b])
            pltpu.make_async_copy(
                dbs[b], out_d.at[cid, pl.ds(base_r + i * CH, CH)],
                dsems[b]).start()
        for i in range(nwb - 2, nwb):
            b = i % 2
            pltpu.make_async_copy(
                dbs[b], out_d.at[cid, pl.ds(base_r + i * CH, CH)],
                dsems[b]).wait()
    for i in range(nwb - NB, nwb):
        b = i % NB
        pltpu.make_async_copy(
            rows[b], out.at[cid, pl.ds(base_r + i * CH, CH)],
            sems[b]).wait()
    _c3.__exit__(None, None, None)


def _sc_agg_body_deg(x, src, dst, zeros, ones16, zeros16, out, out_d, sgi,
                     dgi, *rest):
    rows, zbuf, acc, acc_d, ones_v, dbuf = rest[:NB], rest[NB], rest[NB + 1], rest[NB + 2], rest[NB + 3], rest[NB + 4]
    sems, semi, semz, semd = rest[NB + 5:2 * NB + 5], rest[2 * NB + 5], rest[2 * NB + 6], rest[2 * NB + 7]
    _sc_agg_body(x, src, dst, zeros, ones16, zeros16, (out, out_d), sgi,
                 dgi, rows, zbuf, acc, acc_d, ones_v, dbuf,
                 sems, semi, semz, semd, with_deg=True)


def _sc_agg_body_nodeg(x, src, dst, zeros, out, sgi, dgi, *rest):
    rows, zbuf, acc = rest[:NB], rest[NB], rest[NB + 1]
    sems, semi, semz = rest[NB + 2:2 * NB + 2], rest[2 * NB + 2], rest[2 * NB + 3]
    _sc_agg_body(x, src, dst, zeros, None, None, out, sgi, dgi, rows,
                 zbuf, acc, None, None, None, sems, semi, semz, None,
                 with_deg=False)


_sc_agg_deg = functools.partial(
    pl.kernel,
    mesh=plsc.VectorSubcoreMesh(core_axis_name="c", subcore_axis_name="s"),
    out_type=(jax.ShapeDtypeStruct((NC, NPAD, D), jnp.float32),
              jax.ShapeDtypeStruct((NC, NPAD, DG), jnp.float32)),
    scratch_types=[
        pltpu.VMEM((2, NB, CH), jnp.int32),
        pltpu.VMEM((2, NB, CH), jnp.int32),
    ] + [pltpu.VMEM((CH, D), jnp.float32) for _ in range(NB)] + [
        pltpu.VMEM((16, D), jnp.float32),
        pltpu.VMEM_SHARED((NPAD, D), jnp.float32),
        pltpu.VMEM_SHARED((NPAD, DG), jnp.float32),
        pltpu.VMEM((CH, DG), jnp.float32),
        pltpu.VMEM((CH, DG), jnp.float32),
    ] + [pltpu.SemaphoreType.DMA for _ in range(NB + 3)],
    compiler_params=pltpu.CompilerParams(use_tc_tiling_on_sc=False),
)(_sc_agg_body_deg)

_sc_agg_nodeg = functools.partial(
    pl.kernel,
    mesh=plsc.VectorSubcoreMesh(core_axis_name="c", subcore_axis_name="s"),
    out_type=jax.ShapeDtypeStruct((NC, NPAD, D), jnp.float32),
    scratch_types=[
        pltpu.VMEM((2, NB, CH), jnp.int32),
        pltpu.VMEM((2, NB, CH), jnp.int32),
    ] + [pltpu.VMEM((CH, D), jnp.float32) for _ in range(NB)] + [
        pltpu.VMEM((16, D), jnp.float32),
        pltpu.VMEM_SHARED((NPAD, D), jnp.float32),
    ] + [pltpu.SemaphoreType.DMA for _ in range(NB + 2)],
    compiler_params=pltpu.CompilerParams(use_tc_tiling_on_sc=False),
)(_sc_agg_body_nodeg)


def _self_body(x_ref, b_ref, bias_ref, out_ref):
    out_ref[:] = jnp.dot(
        x_ref[:], b_ref[:], preferred_element_type=jnp.float32) + bias_ref[:]


def _self_matmul(x, b_w, bias):
    br = 1000
    return pl.pallas_call(
        _self_body,
        grid=(N // br,),
        in_specs=[
            pl.BlockSpec((br, D), lambda i: (i, 0)),
            pl.BlockSpec((D, D), lambda i: (0, 0)),
            pl.BlockSpec((1, D), lambda i: (0, 0)),
        ],
        out_specs=pl.BlockSpec((br, D), lambda i: (i, 0)),
        out_shape=jax.ShapeDtypeStruct((N, D), jnp.float32),
    )(x, b_w, bias)


def _comb_body(sums_ref, degs_ref, self_ref, a_ref, out_ref, *, relu):
    s = sums_ref[0] + sums_ref[1]
    deg = jnp.maximum(degs_ref[0, :, :1] + degs_ref[1, :, :1], 1.0)
    y = jnp.dot(s / deg, a_ref[:], preferred_element_type=jnp.float32)
    y = y + self_ref[:]
    if relu:
        y = jnp.maximum(y, 0.0)
    out_ref[:] = y


def _dense_layer(sums, degs, x_self, a_w, relu):
    br = 1000
    return pl.pallas_call(
        functools.partial(_comb_body, relu=relu),
        grid=(N // br,),
        in_specs=[
            pl.BlockSpec((NC, br, D), lambda i: (0, i, 0)),
            pl.BlockSpec((NC, br, DG), lambda i: (0, i, 0)),
            pl.BlockSpec((br, D), lambda i: (i, 0)),
            pl.BlockSpec((D, D), lambda i: (0, 0)),
        ],
        out_specs=pl.BlockSpec((br, D), lambda i: (i, 0)),
        out_shape=jax.ShapeDtypeStruct((N, D), jnp.float32),
    )(sums, degs, x_self, a_w)


def kernel(x, edge_index, W0l, W0r, b0, bn_gamma, bn_beta, bn_mean, bn_var, W1l, W1r, b1):
    src = edge_index[0]
    dst = edge_index[1]

    g = bn_gamma / jnp.sqrt(bn_var + 1e-5)
    c = bn_beta - bn_mean * g
    a0 = W0l.T * g
    b0w = W0r.T * g
    bias0 = (b0 * g + c)[None, :]
    a1 = W1l.T
    b1w = W1r.T
    bias1 = b1[None, :]

    zeros = jnp.zeros((CH, D), jnp.float32)
    ones16 = jnp.ones((CH, DG), jnp.float32)
    zeros16 = jnp.zeros((CH, DG), jnp.float32)

    pad_e = EPAD - E
    pad_src = jnp.arange(pad_e, dtype=jnp.int32) % N
    src_c = jnp.concatenate([src, pad_src]).reshape(TOTCH, CH)
    pad_dst = N + (jnp.arange(pad_e, dtype=jnp.int32) % (NPAD - N))
    dst_c = jnp.concatenate([dst, pad_dst]).reshape(TOTCH, CH)

    self0 = _self_matmul(x, b0w, bias0)
    sums0, degs = _sc_agg_deg(x, src_c, dst_c, zeros, ones16, zeros16)
    h = _dense_layer(sums0, degs, self0, a0, relu=True)
    self1 = _self_matmul(h, b1w, bias1)
    sums1 = _sc_agg_nodeg(h, src_c, dst_c, zeros)
    logit = _dense_layer(sums1, degs, self1, a1, relu=False)

    return (logit, h)


# speedup vs baseline: 2.1977x; 1.0277x over previous
import functools

import jax
import jax.numpy as jnp
from jax import lax
from jax.experimental import pallas as pl
from jax.experimental.pallas import tpu as pltpu
from jax.experimental.pallas import tpu_sc as plsc

N = 10000
E = 320000
D = 128
DG = 16
NPAD = 10240
NC = 2
NS = 16
EPAD = 322560
CH = 80
TOTCH = EPAD // CH
NB = 3
FAST_CID = 0
CPW_FAST = 126
CPW_SLOW = 126
RPS = NPAD // NS


def _sc_agg_body(x, ei, zeros, ones16, zeros16, outs, sgi, dgi,
                 rows, zbuf, acc, acc_d, ones_v, dbuf,
                 sems, semi, semz, semd, *, with_deg):
    if with_deg:
        out, out_d = outs
    else:
        out = outs
    cid = lax.axis_index("c")
    sid = lax.axis_index("s")
    fast = cid == FAST_CID
    cb = jnp.where(fast, NS * CPW_SLOW + sid * CPW_FAST, sid * CPW_SLOW)
    ngrp = jnp.where(fast, CPW_FAST // NB, CPW_SLOW // NB)
    npair = jnp.where(fast, CPW_FAST // (2 * NB), CPW_SLOW // (2 * NB))
    base_r = sid * RPS

    _c1 = jax.named_scope("ph_pre")
    _c1.__enter__()
    pltpu.sync_copy(ei.at[0, pl.ds(cb, NB)], sgi.at[0])
    pltpu.sync_copy(ei.at[1, pl.ds(cb, NB)], dgi.at[0])
    pltpu.make_async_copy(ei.at[0, pl.ds(cb + NB, NB)], sgi.at[1], semi).start()
    pltpu.make_async_copy(ei.at[1, pl.ds(cb + NB, NB)], dgi.at[1], semi).start()
    for b in range(NB):
        pltpu.make_async_copy(x.at[sgi.at[0, b]], rows[b], sems[b]).start()

    pltpu.sync_copy(zeros.at[pl.ds(0, 16)], zbuf)
    if with_deg:
        pltpu.sync_copy(ones16, ones_v)
        pltpu.sync_copy(zeros16, dbuf)
        for i in range(RPS // CH):
            r = base_r + i * CH
            pltpu.make_async_copy(dbuf, acc_d.at[pl.ds(r, CH)], semd).start()
    nz = RPS // 16
    for w in range(0, nz, 8):
        for i in range(w, w + 8):
            r = base_r + i * 16
            pltpu.make_async_copy(zbuf, acc.at[pl.ds(r, 16)], semz).start()
        for i in range(w, w + 8):
            r = base_r + i * 16
            pltpu.make_async_copy(zbuf, acc.at[pl.ds(r, 16)], semz).wait()
    if with_deg:
        for i in range(RPS // CH):
            r = base_r + i * CH
            pltpu.make_async_copy(dbuf, acc_d.at[pl.ds(r, CH)], semd).wait()
    plsc.subcore_barrier()
    _c1.__exit__(None, None, None)
    _c2 = jax.named_scope("ph_edges")
    _c2.__enter__()

    def pair(p, carry):
        for sl in range(2):
            g = 2 * p + sl
            nsl = 1 - sl

            @pl.when(g + 1 < ngrp)
            def _():
                pltpu.make_async_copy(
                    ei.at[0, pl.ds(cb + (g + 1) * NB, NB)], sgi.at[nsl],
                    semi).wait()
                pltpu.make_async_copy(
                    ei.at[1, pl.ds(cb + (g + 1) * NB, NB)], dgi.at[nsl],
                    semi).wait()

            for b in range(NB):
                pltpu.make_async_copy(
                    x.at[sgi.at[sl, b]], rows[b], sems[b]).wait()
                pltpu.sync_copy(rows[b], acc.at[dgi.at[sl, b]], add=True)
                if with_deg:
                    pltpu.sync_copy(ones_v, acc_d.at[dgi.at[sl, b]],
                                    add=True)

                @pl.when(g + 1 < ngrp)
                def _():
                    pltpu.make_async_copy(
                        x.at[sgi.at[nsl, b]], rows[b], sems[b]).start()

            @pl.when(g + 2 < ngrp)
            def _():
                pltpu.make_async_copy(
                    ei.at[0, pl.ds(cb + (g + 2) * NB, NB)], sgi.at[sl],
                    semi).start()
                pltpu.make_async_copy(
                    ei.at[1, pl.ds(cb + (g + 2) * NB, NB)], dgi.at[sl],
                    semi).start()

        return carry

    lax.fori_loop(0, npair, pair, 0)
    plsc.subcore_barrier()
    _c2.__exit__(None, None, None)
    _c3 = jax.named_scope("ph_wb")
    _c3.__enter__()

    nwb = RPS // CH
    for i in range(nwb):
        b = i % NB
        if i >= NB:
            pltpu.make_async_copy(
                rows[b], out.at[cid, pl.ds(base_r + (i - NB) * CH, CH)],
                sems[b]).wait()
        pltpu.sync_copy(acc.at[pl.ds(base_r + i * CH, CH)], rows[b])
        pltpu.make_async_copy(
            rows[b], out.at[cid, pl.ds(base_r + i * CH, CH)],
            sems[b]).start()
    if with_deg:
        dbs = (dbuf, ones_v)
        dsems = (semd, semz)
        for i in range(nwb):
            b = i % 2
            if i >= 2:
                pltpu.make_async_copy(
                    dbs[b], out_d.at[cid, pl.ds(base_r + (i - 2) * CH, CH)],
                    dsems[b]).wait()
            pltpu.sync_copy(acc_d.at[pl.ds(base_r + i * CH, CH)], dbs[b])
            pltpu.make_async_copy(
                dbs[b], out_d.at[cid, pl.ds(base_r + i * CH, CH)],
                dsems[b]).start()
        for i in range(nwb - 2, nwb):
            b = i % 2
            pltpu.make_async_copy(
                dbs[b], out_d.at[cid, pl.ds(base_r + i * CH, CH)],
                dsems[b]).wait()
    for i in range(nwb - NB, nwb):
        b = i % NB
        pltpu.make_async_copy(
            rows[b], out.at[cid, pl.ds(base_r + i * CH, CH)],
            sems[b]).wait()
    _c3.__exit__(None, None, None)


def _sc_agg_body_deg(x, ei, zeros, ones16, zeros16, out, out_d, sgi,
                     dgi, *rest):
    rows, zbuf, acc, acc_d, ones_v, dbuf = rest[:NB], rest[NB], rest[NB + 1], rest[NB + 2], rest[NB + 3], rest[NB + 4]
    sems, semi, semz, semd = rest[NB + 5:2 * NB + 5], rest[2 * NB + 5], rest[2 * NB + 6], rest[2 * NB + 7]
    _sc_agg_body(x, ei, zeros, ones16, zeros16, (out, out_d), sgi,
                 dgi, rows, zbuf, acc, acc_d, ones_v, dbuf,
                 sems, semi, semz, semd, with_deg=True)


def _sc_agg_body_nodeg(x, ei, zeros, out, sgi, dgi, *rest):
    rows, zbuf, acc = rest[:NB], rest[NB], rest[NB + 1]
    sems, semi, semz = rest[NB + 2:2 * NB + 2], rest[2 * NB + 2], rest[2 * NB + 3]
    _sc_agg_body(x, ei, zeros, None, None, out, sgi, dgi, rows,
                 zbuf, acc, None, None, None, sems, semi, semz, None,
                 with_deg=False)


_sc_agg_deg = functools.partial(
    pl.kernel,
    mesh=plsc.VectorSubcoreMesh(core_axis_name="c", subcore_axis_name="s"),
    out_type=(jax.ShapeDtypeStruct((NC, NPAD, D), jnp.float32),
              jax.ShapeDtypeStruct((NC, NPAD, DG), jnp.float32)),
    scratch_types=[
        pltpu.VMEM((2, NB, CH), jnp.int32),
        pltpu.VMEM((2, NB, CH), jnp.int32),
    ] + [pltpu.VMEM((CH, D), jnp.float32) for _ in range(NB)] + [
        pltpu.VMEM((16, D), jnp.float32),
        pltpu.VMEM_SHARED((NPAD, D), jnp.float32),
        pltpu.VMEM_SHARED((NPAD, DG), jnp.float32),
        pltpu.VMEM((CH, DG), jnp.float32),
        pltpu.VMEM((CH, DG), jnp.float32),
    ] + [pltpu.SemaphoreType.DMA for _ in range(NB + 3)],
    compiler_params=pltpu.CompilerParams(use_tc_tiling_on_sc=False),
)(_sc_agg_body_deg)

_sc_agg_nodeg = functools.partial(
    pl.kernel,
    mesh=plsc.VectorSubcoreMesh(core_axis_name="c", subcore_axis_name="s"),
    out_type=jax.ShapeDtypeStruct((NC, NPAD, D), jnp.float32),
    scratch_types=[
        pltpu.VMEM((2, NB, CH), jnp.int32),
        pltpu.VMEM((2, NB, CH), jnp.int32),
    ] + [pltpu.VMEM((CH, D), jnp.float32) for _ in range(NB)] + [
        pltpu.VMEM((16, D), jnp.float32),
        pltpu.VMEM_SHARED((NPAD, D), jnp.float32),
    ] + [pltpu.SemaphoreType.DMA for _ in range(NB + 2)],
    compiler_params=pltpu.CompilerParams(use_tc_tiling_on_sc=False),
)(_sc_agg_body_nodeg)


def _self_body(x_ref, b_ref, bias_ref, out_ref):
    out_ref[:] = jnp.dot(
        x_ref[:], b_ref[:], preferred_element_type=jnp.float32) + bias_ref[:]


def _self_matmul(x, b_w, bias):
    br = 1000
    return pl.pallas_call(
        _self_body,
        grid=(N // br,),
        in_specs=[
            pl.BlockSpec((br, D), lambda i: (i, 0)),
            pl.BlockSpec((D, D), lambda i: (0, 0)),
            pl.BlockSpec((1, D), lambda i: (0, 0)),
        ],
        out_specs=pl.BlockSpec((br, D), lambda i: (i, 0)),
        out_shape=jax.ShapeDtypeStruct((N, D), jnp.float32),
    )(x, b_w, bias)


def _comb_body(sums_ref, degs_ref, self_ref, a_ref, out_ref, *, relu):
    s = sums_ref[0] + sums_ref[1]
    deg = jnp.maximum(degs_ref[0, :, :1] + degs_ref[1, :, :1], 1.0)
    y = jnp.dot(s / deg, a_ref[:], preferred_element_type=jnp.float32)
    y = y + self_ref[:]
    if relu:
        y = jnp.maximum(y, 0.0)
    out_ref[:] = y


def _dense_layer(sums, degs, x_self, a_w, relu):
    br = 1000
    return pl.pallas_call(
        functools.partial(_comb_body, relu=relu),
        grid=(N // br,),
        in_specs=[
            pl.BlockSpec((NC, br, D), lambda i: (0, i, 0)),
            pl.BlockSpec((NC, br, DG), lambda i: (0, i, 0)),
            pl.BlockSpec((br, D), lambda i: (i, 0)),
            pl.BlockSpec((D, D), lambda i: (0, 0)),
        ],
        out_specs=pl.BlockSpec((br, D), lambda i: (i, 0)),
        out_shape=jax.ShapeDtypeStruct((N, D), jnp.float32),
    )(sums, degs, x_self, a_w)


def kernel(x, edge_index, W0l, W0r, b0, bn_gamma, bn_beta, bn_mean, bn_var, W1l, W1r, b1):
    src = edge_index[0]
    dst = edge_index[1]

    g = bn_gamma / jnp.sqrt(bn_var + 1e-5)
    c = bn_beta - bn_mean * g
    a0 = W0l.T * g
    b0w = W0r.T * g
    bias0 = (b0 * g + c)[None, :]
    a1 = W1l.T
    b1w = W1r.T
    bias1 = b1[None, :]

    zeros = jnp.zeros((CH, D), jnp.float32)
    ones16 = jnp.ones((CH, DG), jnp.float32)
    zeros16 = jnp.zeros((CH, DG), jnp.float32)

    pad_e = EPAD - E
    pad_src = jnp.arange(pad_e, dtype=jnp.int32) % N
    pad_dst = N + (jnp.arange(pad_e, dtype=jnp.int32) % (NPAD - N))
    ei = jnp.concatenate(
        [edge_index, jnp.stack([pad_src, pad_dst])], axis=1
    ).reshape(2, TOTCH, CH)

    self0 = _self_matmul(x, b0w, bias0)
    sums0, degs = _sc_agg_deg(x, ei, zeros, ones16, zeros16)
    h = _dense_layer(sums0, degs, self0, a0, relu=True)
    self1 = _self_matmul(h, b1w, bias1)
    sums1 = _sc_agg_nodeg(h, ei, zeros)
    logit = _dense_layer(sums1, degs, self1, a1, relu=False)

    return (logit, h)
